# baseline (speedup 1.0000x reference)
import functools

import jax
import jax.numpy as jnp
from jax import lax
from jax.experimental import pallas as pl
from jax.experimental.pallas import tpu as pltpu
from jax.experimental.pallas import tpu_sc as plsc

E = 64
K = 8
COEF = 0.01
TB = 1024
CHUNKS = (16, 8, 8)
NBLK = sum(CHUNKS)
NC, NS, L = 2, 16, 16
NW = NC * NS
T_TOTAL = NBLK * TB
MASK6 = ~63
SENT = -2147483648
AUXN = TB * K


def _tc_body(x_ref, w_ref, lt_ref, ps_ref, acc_ref):
    i = pl.program_id(0)
    n = pl.num_programs(0)
    x = x_ref[...]
    w = w_ref[...]
    lt = lax.dot_general(w, x, (((1,), (1,)), ((), ())),
                         preferred_element_type=jnp.float32)

    @pl.when(i == 0)
    def _init():
        acc_ref[...] = jnp.zeros_like(acc_ref)

    colmax = jnp.max(lt, axis=0, keepdims=True)
    ex = jnp.exp(lt - colmax)
    recip = 1.0 / jnp.sum(ex, axis=0, keepdims=True)
    acc_ref[...] = acc_ref[...] + lax.dot_general(
        recip, ex, (((1,), (1,)), ((), ())),
        preferred_element_type=jnp.float32)
    lt_ref[...] = lt[None]

    @pl.when(i == n - 1)
    def _fin():
        ps_ref[...] = jnp.concatenate(
            [acc_ref[...], jnp.zeros((1, 128 - E), jnp.float32)], axis=1)


def _ord(u):
    return u ^ (lax.shift_right_arithmetic(u, 31) & 0x7FFFFFFF)


def _make_route_body(cb, with_aux=False):
    ct = cb * TB
    tpw = ct // NW
    per_blk = TB // tpw

    def _sc_route_body(*refs):
        if with_aux:
            (lt_hbm, ti0_hbm, ti1_hbm, ps0, ps1, ps2,
             tw_hbm, ti_hbm, auxp_hbm,
             lt_v, tw_v, ti_v, tiv, sv, aux_v) = refs
        else:
            lt_hbm, tw_hbm, ti_hbm, lt_v, tw_v, ti_v = refs
        c = lax.axis_index("c")
        sx = lax.axis_index("s")
        wid = sx * NC + c
        b = wid // per_blk
        q = wid % per_blk

        pltpu.sync_copy(lt_hbm.at[b, :, pl.ds(q * tpw, tpw)], lt_v)

        def group(g, carry):
            base = g * L
            t = [jnp.full((L,), SENT, jnp.int32) for _ in range(K)]
            for e in range(E):
                v = lt_v[e, pl.ds(base, L)]
                u = lax.bitcast_convert_type(v, jnp.int32)
                cur = (_ord(u) & MASK6) | (63 - e)
                for j in range(K):
                    hi = lax.max(t[j], cur)
                    cur = lax.min(t[j], cur)
                    t[j] = hi
            exs = []
            idxs = []
            v0 = None
            for j in range(K):
                aj = 63 - (t[j] & 63)
                vj = lax.bitcast_convert_type(_ord(t[j] & MASK6), jnp.float32)
                if j == 0:
                    v0 = vj
                idxs.append(aj)
                exs.append(jnp.exp(vj - v0))
            tot = exs[0]
            for j in range(1, K):
                tot = tot + exs[j]
            inv = 1.0 / tot
            for j in range(K):
                tw_v[pl.ds(j * tpw + base, L)] = exs[j] * inv
                ti_v[pl.ds(j * tpw + base, L)] = idxs[j]
            return carry

        lax.fori_loop(0, tpw // L, group, jnp.int32(0))

        for j in range(K):
            pltpu.sync_copy(tw_v.at[pl.ds(j * tpw, tpw)],
                            tw_hbm.at[pl.ds(j * ct + wid * tpw, tpw)])
            pltpu.sync_copy(ti_v.at[pl.ds(j * tpw, tpw)],
                            ti_hbm.at[pl.ds(j * ct + wid * tpw, tpw)])

        if with_aux:
            n0 = K * CHUNKS[0] * TB
            share = (n0 + K * CHUNKS[1] * TB) // NW
            cut = n0 // share
            start = wid * share
            rem = n0 - cut * share

            @pl.when(wid < cut)
            def _():
                pltpu.sync_copy(ti0_hbm.at[pl.ds(start, share)], tiv)

            @pl.when(wid == cut)
            def _():
                pltpu.sync_copy(ti0_hbm.at[pl.ds(cut * share, rem)],
                                tiv.at[pl.ds(0, rem)])
                pltpu.sync_copy(ti1_hbm.at[pl.ds(0, share - rem)],
                                tiv.at[pl.ds(rem, share - rem)])

            @pl.when(wid > cut)
            def _():
                pltpu.sync_copy(ti1_hbm.at[pl.ds(start - n0, share)], tiv)

            for i, ps_h in enumerate((ps0, ps1, ps2)):
                pltpu.sync_copy(ps_h.at[0], sv.at[pl.ds(i * 128, 128)])
            scale = COEF * E / (float(T_TOTAL) * float(T_TOTAL))
            s_tab = []
            for p in range(4):
                tab = sv[pl.ds(p * L, L)]
                for i in range(1, len(CHUNKS)):
                    tab = tab + sv[pl.ds(i * 128 + p * L, L)]
                s_tab.append(tab * scale)

            def lookup(iv, acc):
                p = lax.shift_right_logical(iv, 4)
                wi = iv & 15
                gv = jnp.take(s_tab[3], wi, mode="fill")
                for qq in range(2, -1, -1):
                    gv = jnp.where(p == qq,
                                   jnp.take(s_tab[qq], wi, mode="fill"), gv)
                return acc + gv

            def step_h(i, acc):
                return lookup(tiv[pl.ds(i * L, L)], acc)

            def step_l(i, acc):
                return lookup(ti_v[pl.ds(i * L, L)], acc)

            acc = lax.fori_loop(0, share // L, step_h,
                                jnp.zeros((L,), jnp.float32))
            acc = lax.fori_loop(0, (K * tpw) // L, step_l, acc)
            aux_v[...] = acc
            pltpu.sync_copy(aux_v, auxp_hbm.at[wid])

    return _sc_route_body, ct, tpw


def _sc_aux_body(*refs):
    nch = len(CHUNKS)
    tis = refs[:nch]
    pss = refs[nch:2 * nch]
    auxp_hbm = refs[2 * nch]
    tiv, sv, aux_v = refs[2 * nch + 1:]
    c = lax.axis_index("c")
    sx = lax.axis_index("s")
    wid = sx * NC + c
    starts = []
    acc0 = 0
    for cb in CHUNKS:
        starts.append(acc0)
        acc0 += cb

    for i, ti_h in enumerate(tis):
        @pl.when(jnp.logical_and(wid >= starts[i],
                                 wid < starts[i] + CHUNKS[i]))
        def _(ti_h=ti_h, st=starts[i]):
            pltpu.sync_copy(ti_h.at[pl.ds((wid - st) * AUXN, AUXN)], tiv)

    for i, ps_h in enumerate(pss):
        pltpu.sync_copy(ps_h.at[0], sv.at[pl.ds(i * 128, 128)])
    scale = COEF * E / (float(T_TOTAL) * float(T_TOTAL))
    s_tab = []
    for p in range(4):
        tab = sv[pl.ds(p * L, L)]
        for i in range(1, len(CHUNKS)):
            tab = tab + sv[pl.ds(i * 128 + p * L, L)]
        s_tab.append(tab * scale)

    def step(i, acc):
        iv = tiv[pl.ds(i * L, L)]
        p = lax.shift_right_logical(iv, 4)
        wi = iv & 15
        gv = jnp.take(s_tab[3], wi, mode="fill")
        for q in range(2, -1, -1):
            gv = jnp.where(p == q, jnp.take(s_tab[q], wi, mode="fill"), gv)
        return acc + gv

    acc = lax.fori_loop(0, AUXN // L, step, jnp.zeros((L,), jnp.float32))
    aux_v[...] = acc
    pltpu.sync_copy(aux_v, auxp_hbm.at[wid])


@jax.jit
def kernel(hidden_states, gate_w):
    t, h = hidden_states.shape
    mesh = plsc.VectorSubcoreMesh(core_axis_name="c", subcore_axis_name="s")
    scp = pltpu.CompilerParams(use_tc_tiling_on_sc=True)

    tws, tis, pss = [], [], []
    tifs = []
    blk0 = 0
    auxp = None
    for ci, cb in enumerate(CHUNKS):
        last = ci == len(CHUNKS) - 1
        body, ct, tpw = _make_route_body(cb, with_aux=last)
        share = (K * (CHUNKS[0] + CHUNKS[1]) * TB) // NW
        out_type = [jax.ShapeDtypeStruct((K * ct,), jnp.float32),
                    jax.ShapeDtypeStruct((K * ct,), jnp.int32)]
        scratch = [pltpu.VMEM((E, tpw), jnp.float32),
                   pltpu.VMEM((K * tpw,), jnp.float32),
                   pltpu.VMEM((K * tpw,), jnp.int32)]
        if last:
            out_type.append(jax.ShapeDtypeStruct((NW, L), jnp.float32))
            scratch += [pltpu.VMEM((share,), jnp.int32),
                        pltpu.VMEM((len(CHUNKS) * 128,), jnp.float32),
                        pltpu.VMEM((L,), jnp.float32)]
        sc_route = functools.partial(
            pl.kernel,
            mesh=mesh,
            compiler_params=scp,
            out_type=out_type,
            scratch_types=scratch,
        )(body)

        lt3, ps = pl.pallas_call(
            _tc_body,
            grid=(cb,),
            in_specs=[pl.BlockSpec((TB, h), lambda i, b0=blk0: (b0 + i, 0)),
                      pl.BlockSpec((E, h), lambda i: (0, 0))],
            out_specs=[pl.BlockSpec((1, E, TB), lambda i: (i, 0, 0)),
                       pl.BlockSpec((1, 128), lambda i: (0, 0))],
            out_shape=[jax.ShapeDtypeStruct((cb, E, TB), jnp.float32),
                       jax.ShapeDtypeStruct((1, 128), jnp.float32)],
            scratch_shapes=[pltpu.VMEM((1, E), jnp.float32)],
        )(hidden_states, gate_w)
        pss.append(ps)
        if last:
            twf, tif, auxp = sc_route(lt3, tifs[0], tifs[1],
                                      pss[0], pss[1], pss[2])
        else:
            twf, tif = sc_route(lt3)
        tifs.append(tif)
        tws.append(twf.reshape(K, ct))
        tis.append(tif.reshape(K, ct))
        blk0 += cb

    tw = jnp.concatenate(tws, axis=1).T
    ti = jnp.concatenate(tis, axis=1).T
    return tw, ti, jnp.sum(auxp)


# speedup vs baseline: 1.0025x; 1.0025x over previous
import functools

import jax
import jax.numpy as jnp
from jax import lax
from jax.experimental import pallas as pl
from jax.experimental.pallas import tpu as pltpu
from jax.experimental.pallas import tpu_sc as plsc

E = 64
K = 8
COEF = 0.01
TB = 1024
CHUNKS = (16, 8, 8)
NBLK = sum(CHUNKS)
NC, NS, L = 2, 16, 16
NW = NC * NS
T_TOTAL = NBLK * TB
MASK6 = ~63
SENT = -2147483648


def _tc_body(x_ref, w_ref, lt_ref, ps_ref, acc_ref):
    i = pl.program_id(0)
    n = pl.num_programs(0)
    x = x_ref[...]
    w = w_ref[...]
    lt = lax.dot_general(w, x, (((1,), (1,)), ((), ())),
                         preferred_element_type=jnp.float32)

    @pl.when(i == 0)
    def _init():
        acc_ref[...] = jnp.zeros_like(acc_ref)

    colmax = jnp.max(lt, axis=0, keepdims=True)
    ex = jnp.exp(lt - colmax)
    recip = 1.0 / jnp.sum(ex, axis=0, keepdims=True)
    acc_ref[...] = acc_ref[...] + lax.dot_general(
        recip, ex, (((1,), (1,)), ((), ())),
        preferred_element_type=jnp.float32)
    lt_ref[...] = lt[None]

    @pl.when(i == n - 1)
    def _fin():
        ps_ref[...] = jnp.concatenate(
            [acc_ref[...], jnp.zeros((1, 128 - E), jnp.float32)], axis=1)


def _ord(u):
    return u ^ (lax.shift_right_arithmetic(u, 31) & 0x7FFFFFFF)


def _make_route_body(cb, with_aux=False):
    ct = cb * TB
    tpw = ct // NW
    per_blk = TB // tpw

    def _sc_route_body(*refs):
        if with_aux:
            (lt_hbm, ti0_hbm, ti1_hbm, ps0, ps1, ps2,
             tw_hbm, ti_hbm, auxp_hbm,
             lt_v, tw_v, ti_v, tiv, sv, aux_v) = refs
        else:
            lt_hbm, tw_hbm, ti_hbm, lt_v, tw_v, ti_v = refs
        c = lax.axis_index("c")
        sx = lax.axis_index("s")
        wid = sx * NC + c
        b = wid // per_blk
        q = wid % per_blk

        pltpu.sync_copy(lt_hbm.at[b, :, pl.ds(q * tpw, tpw)], lt_v)

        def group(g, carry):
            base = g * L
            t = [jnp.full((L,), SENT, jnp.int32) for _ in range(K)]
            for e in range(E):
                v = lt_v[e, pl.ds(base, L)]
                u = lax.bitcast_convert_type(v, jnp.int32)
                cur = (_ord(u) & MASK6) | (63 - e)
                for j in range(K):
                    hi = lax.max(t[j], cur)
                    cur = lax.min(t[j], cur)
                    t[j] = hi
            exs = []
            idxs = []
            v0 = None
            for j in range(K):
                aj = 63 - (t[j] & 63)
                vj = lax.bitcast_convert_type(_ord(t[j] & MASK6), jnp.float32)
                if j == 0:
                    v0 = vj
                idxs.append(aj)
                exs.append(jnp.exp(vj - v0))
            tot = exs[0]
            for j in range(1, K):
                tot = tot + exs[j]
            inv = 1.0 / tot
            for j in range(K):
                tw_v[pl.ds(j * tpw + base, L)] = exs[j] * inv
                ti_v[pl.ds(j * tpw + base, L)] = idxs[j]
            return carry

        lax.fori_loop(0, tpw // L, group, jnp.int32(0))

        for j in range(K):
            pltpu.sync_copy(tw_v.at[pl.ds(j * tpw, tpw)],
                            tw_hbm.at[pl.ds(j * ct + wid * tpw, tpw)])
            pltpu.sync_copy(ti_v.at[pl.ds(j * tpw, tpw)],
                            ti_hbm.at[pl.ds(j * ct + wid * tpw, tpw)])

        if with_aux:
            n0 = K * CHUNKS[0] * TB
            share = (n0 + K * CHUNKS[1] * TB) // NW
            cut = n0 // share
            start = wid * share
            rem = n0 - cut * share

            @pl.when(wid < cut)
            def _():
                pltpu.sync_copy(ti0_hbm.at[pl.ds(start, share)], tiv)

            @pl.when(wid == cut)
            def _():
                pltpu.sync_copy(ti0_hbm.at[pl.ds(cut * share, rem)],
                                tiv.at[pl.ds(0, rem)])
                pltpu.sync_copy(ti1_hbm.at[pl.ds(0, share - rem)],
                                tiv.at[pl.ds(rem, share - rem)])

            @pl.when(wid > cut)
            def _():
                pltpu.sync_copy(ti1_hbm.at[pl.ds(start - n0, share)], tiv)

            for i, ps_h in enumerate((ps0, ps1, ps2)):
                pltpu.sync_copy(ps_h.at[0], sv.at[pl.ds(i * 128, 128)])
            scale = COEF * E / (float(T_TOTAL) * float(T_TOTAL))
            s_tab = []
            for p in range(4):
                tab = sv[pl.ds(p * L, L)]
                for i in range(1, len(CHUNKS)):
                    tab = tab + sv[pl.ds(i * 128 + p * L, L)]
                s_tab.append(tab * scale)

            def lookup(iv, acc):
                p = lax.shift_right_logical(iv, 4)
                wi = iv & 15
                gv = jnp.take(s_tab[3], wi, mode="fill")
                for qq in range(2, -1, -1):
                    gv = jnp.where(p == qq,
                                   jnp.take(s_tab[qq], wi, mode="fill"), gv)
                return acc + gv

            def step_h(i, acc):
                return lookup(tiv[pl.ds(i * L, L)], acc)

            def step_l(i, acc):
                return lookup(ti_v[pl.ds(i * L, L)], acc)

            acc = lax.fori_loop(0, share // L, step_h,
                                jnp.zeros((L,), jnp.float32))
            acc = lax.fori_loop(0, (K * tpw) // L, step_l, acc)
            aux_v[...] = acc
            pltpu.sync_copy(aux_v, auxp_hbm.at[wid])

    return _sc_route_body, ct, tpw


@jax.jit
def kernel(hidden_states, gate_w):
    t, h = hidden_states.shape
    mesh = plsc.VectorSubcoreMesh(core_axis_name="c", subcore_axis_name="s")
    scp = pltpu.CompilerParams(use_tc_tiling_on_sc=True)

    tws, tis, pss = [], [], []
    tifs = []
    blk0 = 0
    auxp = None
    for ci, cb in enumerate(CHUNKS):
        last = ci == len(CHUNKS) - 1
        body, ct, tpw = _make_route_body(cb, with_aux=last)
        share = (K * (CHUNKS[0] + CHUNKS[1]) * TB) // NW
        out_type = [jax.ShapeDtypeStruct((K * ct,), jnp.float32),
                    jax.ShapeDtypeStruct((K * ct,), jnp.int32)]
        scratch = [pltpu.VMEM((E, tpw), jnp.float32),
                   pltpu.VMEM((K * tpw,), jnp.float32),
                   pltpu.VMEM((K * tpw,), jnp.int32)]
        if last:
            out_type.append(jax.ShapeDtypeStruct((NW, L), jnp.float32))
            scratch += [pltpu.VMEM((share,), jnp.int32),
                        pltpu.VMEM((len(CHUNKS) * 128,), jnp.float32),
                        pltpu.VMEM((L,), jnp.float32)]
        sc_route = functools.partial(
            pl.kernel,
            mesh=mesh,
            compiler_params=scp,
            out_type=out_type,
            scratch_types=scratch,
        )(body)

        lt3, ps = pl.pallas_call(
            _tc_body,
            grid=(cb,),
            in_specs=[pl.BlockSpec((TB, h), lambda i, b0=blk0: (b0 + i, 0)),
                      pl.BlockSpec((E, h), lambda i: (0, 0))],
            out_specs=[pl.BlockSpec((1, E, TB), lambda i: (i, 0, 0)),
                       pl.BlockSpec((1, 128), lambda i: (0, 0))],
            out_shape=[jax.ShapeDtypeStruct((cb, E, TB), jnp.float32),
                       jax.ShapeDtypeStruct((1, 128), jnp.float32)],
            scratch_shapes=[pltpu.VMEM((1, E), jnp.float32)],
        )(hidden_states, gate_w)
        pss.append(ps)
        if last:
            twf, tif, auxp = sc_route(lt3, tifs[0], tifs[1],
                                      pss[0], pss[1], pss[2])
        else:
            twf, tif = sc_route(lt3)
        tifs.append(tif)
        tws.append(twf.reshape(K, ct))
        tis.append(tif.reshape(K, ct))
        blk0 += cb

    tw = jnp.concatenate(tws, axis=1).T
    ti = jnp.concatenate(tis, axis=1).T
    return tw, ti, jnp.sum(auxp)
